# baseline (speedup 1.0000x reference)
import functools

import jax
import jax.numpy as jnp
from jax import lax
from jax.experimental import pallas as pl
from jax.experimental.pallas import tpu as pltpu
from jax.experimental.pallas import tpu_sc as plsc

_BATCH = 4096
_SEQ = 200
_D = 64
_DP = 128
_N = _BATCH * _SEQ
_NC, _NS = 2, 16
_NW = _NC * _NS
_BPW = _BATCH // _NW
_ROWS_PER_W = _BPW * _SEQ

_mesh = plsc.VectorSubcoreMesh(core_axis_name="c", subcore_axis_name="s")


@functools.partial(
    pl.kernel,
    mesh=_mesh,
    out_type=jax.ShapeDtypeStruct((_BATCH, _SEQ, _D), jnp.float32),
    scratch_types=[
        pltpu.VMEM((_ROWS_PER_W,), jnp.int32),
        pltpu.VMEM((_SEQ, _DP), jnp.float32),
        pltpu.VMEM((_SEQ, _DP), jnp.float32),
        pltpu.SemaphoreType.DMA,
        pltpu.SemaphoreType.DMA,
    ],
    compiler_params=pltpu.CompilerParams(use_tc_tiling_on_sc=False),
)
def _embed_gather(table_hbm, idx_hbm, out_hbm, idx_v, rows0, rows1, sem0, sem1):
    wid = lax.axis_index("s") * _NC + lax.axis_index("c")
    b0 = wid * _BPW
    pltpu.sync_copy(idx_hbm.at[pl.ds(b0 * _SEQ, _ROWS_PER_W)], idx_v)

    def fire(rows, sem, b):
        off = b * _SEQ
        pltpu.async_copy(table_hbm.at[idx_v.at[pl.ds(off, 128)]],
                         rows.at[pl.ds(0, 128)], sem)
        pltpu.async_copy(table_hbm.at[idx_v.at[pl.ds(off + 128, 72)]],
                         rows.at[pl.ds(128, 72)], sem)

    def drain(rows, sem):
        pltpu.make_async_copy(table_hbm.at[pl.ds(0, _SEQ)], rows, sem).wait()

    def store(rows, b):
        pltpu.sync_copy(rows.at[:, pl.ds(0, _D)], out_hbm.at[b0 + b])

    fire(rows0, sem0, 0)

    def pair(t, carry):
        g0 = 2 * t
        fire(rows1, sem1, g0 + 1)
        drain(rows0, sem0)
        store(rows0, g0)

        @pl.when(t < _BPW // 2 - 1)
        def _():
            fire(rows0, sem0, g0 + 2)

        drain(rows1, sem1)
        store(rows1, g0 + 1)
        return carry

    lax.fori_loop(0, _BPW // 2, pair, 0)


def kernel(token_ids, table):
    idx = token_ids.reshape(-1).astype(jnp.int32)
    table_p = jnp.pad(table, ((0, 0), (0, _DP - _D)))
    return _embed_gather(table_p, idx)


# speedup vs baseline: 1.0126x; 1.0126x over previous
import functools

import jax
import jax.numpy as jnp
from jax import lax
from jax.experimental import pallas as pl
from jax.experimental.pallas import tpu as pltpu
from jax.experimental.pallas import tpu_sc as plsc

_BATCH = 4096
_SEQ = 200
_D = 64
_N = _BATCH * _SEQ
_NC, _NS = 2, 16
_NW = _NC * _NS
_BPW = _BATCH // _NW
_ROWS_PER_W = _BPW * _SEQ

_mesh = plsc.VectorSubcoreMesh(core_axis_name="c", subcore_axis_name="s")


@functools.partial(
    pl.kernel,
    mesh=_mesh,
    out_type=jax.ShapeDtypeStruct((_BATCH, _SEQ, _D), jnp.float32),
    scratch_types=[
        pltpu.VMEM((_ROWS_PER_W,), jnp.int32),
        pltpu.VMEM((_SEQ, _D), jnp.float32),
        pltpu.VMEM((_SEQ, _D), jnp.float32),
        pltpu.SemaphoreType.DMA,
        pltpu.SemaphoreType.DMA,
    ],
    compiler_params=pltpu.CompilerParams(use_tc_tiling_on_sc=False),
)
def _embed_gather(table_hbm, idx_hbm, out_hbm, idx_v, rows0, rows1, sem0, sem1):
    wid = lax.axis_index("s") * _NC + lax.axis_index("c")
    b0 = wid * _BPW
    pltpu.sync_copy(idx_hbm.at[pl.ds(b0 * _SEQ, _ROWS_PER_W)], idx_v)

    def fire(rows, sem, b):
        off = b * _SEQ
        pltpu.async_copy(table_hbm.at[idx_v.at[pl.ds(off, 128)]],
                         rows.at[pl.ds(0, 128)], sem)
        pltpu.async_copy(table_hbm.at[idx_v.at[pl.ds(off + 128, 72)]],
                         rows.at[pl.ds(128, 72)], sem)

    def drain(rows, sem):
        pltpu.make_async_copy(table_hbm.at[pl.ds(0, _SEQ)], rows, sem).wait()

    def store(rows, b):
        pltpu.sync_copy(rows, out_hbm.at[b0 + b])

    fire(rows0, sem0, 0)

    def pair(t, carry):
        g0 = 2 * t
        fire(rows1, sem1, g0 + 1)
        drain(rows0, sem0)
        store(rows0, g0)

        @pl.when(t < _BPW // 2 - 1)
        def _():
            fire(rows0, sem0, g0 + 2)

        drain(rows1, sem1)
        store(rows1, g0 + 1)
        return carry

    lax.fori_loop(0, _BPW // 2, pair, 0)


def kernel(token_ids, table):
    idx = token_ids.reshape(-1).astype(jnp.int32)
    return _embed_gather(table, idx)


# speedup vs baseline: 1.2352x; 1.2199x over previous
import functools

import jax
import jax.numpy as jnp
from jax import lax
from jax.experimental import pallas as pl
from jax.experimental.pallas import tpu as pltpu
from jax.experimental.pallas import tpu_sc as plsc

_BATCH = 4096
_SEQ = 200
_D = 64
_DP = 128
_N = _BATCH * _SEQ
_NC, _NS = 2, 16
_NW = _NC * _NS
_BPW = _BATCH // _NW
_ROWS_PER_W = _BPW * _SEQ

_mesh = plsc.VectorSubcoreMesh(core_axis_name="c", subcore_axis_name="s")


@functools.partial(
    pl.kernel,
    mesh=_mesh,
    out_type=jax.ShapeDtypeStruct((_N, _DP), jnp.float32),
    scratch_types=[
        pltpu.VMEM((_ROWS_PER_W,), jnp.int32),
        pltpu.VMEM((_SEQ, _DP), jnp.float32),
        pltpu.VMEM((_SEQ, _DP), jnp.float32),
        pltpu.SemaphoreType.DMA,
        pltpu.SemaphoreType.DMA,
    ],
    compiler_params=pltpu.CompilerParams(use_tc_tiling_on_sc=True),
)
def _embed_gather(table_hbm, idx_hbm, out_hbm, idx_v, rows0, rows1, sem0, sem1):
    wid = lax.axis_index("s") * _NC + lax.axis_index("c")
    b0 = wid * _BPW
    pltpu.sync_copy(idx_hbm.at[pl.ds(b0 * _SEQ, _ROWS_PER_W)], idx_v)

    def fire(rows, sem, b):
        off = b * _SEQ
        pltpu.async_copy(table_hbm.at[idx_v.at[pl.ds(off, 128)]],
                         rows.at[pl.ds(0, 128)], sem)
        pltpu.async_copy(table_hbm.at[idx_v.at[pl.ds(off + 128, 72)]],
                         rows.at[pl.ds(128, 72)], sem)

    def drain(rows, sem):
        pltpu.make_async_copy(table_hbm.at[pl.ds(0, _SEQ)], rows, sem).wait()

    def store(rows, b):
        pltpu.sync_copy(rows, out_hbm.at[pl.ds((b0 + b) * _SEQ, _SEQ)])

    fire(rows0, sem0, 0)

    def pair(t, carry):
        g0 = 2 * t
        fire(rows1, sem1, g0 + 1)
        drain(rows0, sem0)
        store(rows0, g0)

        @pl.when(t < _BPW // 2 - 1)
        def _():
            fire(rows0, sem0, g0 + 2)

        drain(rows1, sem1)
        store(rows1, g0 + 1)
        return carry

    lax.fori_loop(0, _BPW // 2, pair, 0)


def kernel(token_ids, table):
    idx = token_ids.reshape(-1).astype(jnp.int32)
    table_p = jnp.pad(table, ((0, 0), (0, _DP - _D)))
    out_p = _embed_gather(table_p, idx)
    return out_p[:, : _D].reshape(_BATCH, _SEQ, _D)


# speedup vs baseline: 1.2372x; 1.0016x over previous
import functools

import jax
import jax.numpy as jnp
from jax import lax
from jax.experimental import pallas as pl
from jax.experimental.pallas import tpu as pltpu
from jax.experimental.pallas import tpu_sc as plsc

_BATCH = 4096
_SEQ = 200
_D = 64
_DP = 128
_N = _BATCH * _SEQ
_NC, _NS = 2, 16
_NW = _NC * _NS
_BPW = _BATCH // _NW
_ROWS_PER_W = _BPW * _SEQ

_mesh = plsc.VectorSubcoreMesh(core_axis_name="c", subcore_axis_name="s")


@functools.partial(
    pl.kernel,
    mesh=_mesh,
    out_type=jax.ShapeDtypeStruct((_N, _DP), jnp.float32),
    scratch_types=[
        pltpu.VMEM((_ROWS_PER_W,), jnp.int32),
        pltpu.VMEM((_SEQ, _DP), jnp.float32),
        pltpu.VMEM((_SEQ, _DP), jnp.float32),
        pltpu.SemaphoreType.DMA,
        pltpu.SemaphoreType.DMA,
        pltpu.SemaphoreType.DMA,
        pltpu.SemaphoreType.DMA,
    ],
    compiler_params=pltpu.CompilerParams(use_tc_tiling_on_sc=True),
)
def _embed_gather(table_hbm, idx_hbm, out_hbm, idx_v, rows0, rows1, sem0, sem1,
                  ssem0, ssem1):
    wid = lax.axis_index("s") * _NC + lax.axis_index("c")
    b0 = wid * _BPW
    pltpu.sync_copy(idx_hbm.at[pl.ds(b0 * _SEQ, _ROWS_PER_W)], idx_v)

    def fire(rows, sem, b):
        off = b * _SEQ
        pltpu.async_copy(table_hbm.at[idx_v.at[pl.ds(off, 128)]],
                         rows.at[pl.ds(0, 128)], sem)
        pltpu.async_copy(table_hbm.at[idx_v.at[pl.ds(off + 128, 72)]],
                         rows.at[pl.ds(128, 72)], sem)

    def drain(rows, sem):
        pltpu.make_async_copy(table_hbm.at[pl.ds(0, _SEQ)], rows, sem).wait()

    def store(rows, ssem, b):
        pltpu.async_copy(rows, out_hbm.at[pl.ds((b0 + b) * _SEQ, _SEQ)], ssem)

    def drain_store(rows, ssem):
        pltpu.make_async_copy(table_hbm.at[pl.ds(0, _SEQ)], rows, ssem).wait()

    fire(rows0, sem0, 0)

    def pair(t, carry):
        g0 = 2 * t
        fire(rows1, sem1, g0 + 1)
        drain(rows0, sem0)
        store(rows0, ssem0, g0)

        @pl.when(t < _BPW // 2 - 1)
        def _():
            drain_store(rows0, ssem0)
            fire(rows0, sem0, g0 + 2)

        drain(rows1, sem1)

        @pl.when(t >= 1)
        def _():
            drain_store(rows1, ssem1)

        store(rows1, ssem1, g0 + 1)
        return carry

    lax.fori_loop(0, _BPW // 2, pair, 0)
    drain_store(rows1, ssem1)


def kernel(token_ids, table):
    idx = token_ids.reshape(-1).astype(jnp.int32)
    table_p = jnp.pad(table, ((0, 0), (0, _DP - _D)))
    out_p = _embed_gather(table_p, idx)
    return out_p[:, : _D].reshape(_BATCH, _SEQ, _D)
